# baseline (speedup 1.0000x reference)
import functools

import jax
import jax.numpy as jnp
import numpy as np
from jax.experimental import pallas as pl
from jax.experimental.pallas import tpu as pltpu

B, S, H = 1, 2048, 2048
HQ, HKV, D = 16, 4, 128
E, I = 16, 768
EPS = 1e-6
ROPE_THETA = 10000.0
NEG = float(np.finfo(np.float32).min)

HP = None

BT_QKV = 256


def _qkv_kernel(x_ref, wq_ref, wk_ref, wv_ref, qs_ref, ks_ref, cos_ref,
                sin_ref, q_out, k_out, v_out):
    xb = x_ref[...]
    cos = cos_ref[...][:, None, :]
    sin = sin_ref[...][:, None, :]

    def norm_rope(y, nh, scale):
        y = y.reshape(BT_QKV, nh, D)
        var = jnp.mean(y * y, axis=-1, keepdims=True)
        y = y * jax.lax.rsqrt(var + EPS) * scale.reshape(1, 1, D)
        half = D // 2
        rot = jnp.concatenate([-y[..., half:], y[..., :half]], axis=-1)
        y = y * cos + rot * sin
        return y.reshape(BT_QKV, nh * D)

    q = jnp.dot(xb, wq_ref[...], preferred_element_type=jnp.float32,
                precision=HP)
    q_out[...] = norm_rope(q, HQ, qs_ref[...])
    k = jnp.dot(xb, wk_ref[...], preferred_element_type=jnp.float32,
                precision=HP)
    k_out[...] = norm_rope(k, HKV, ks_ref[...])
    v_out[...] = jnp.dot(xb, wv_ref[...], preferred_element_type=jnp.float32,
                         precision=HP)


def _qkv(x, wq, wk, wv, q_scale, k_scale, cos, sin):
    nb = S // BT_QKV
    return pl.pallas_call(
        _qkv_kernel,
        grid=(nb,),
        in_specs=[
            pl.BlockSpec((BT_QKV, H), lambda i: (i, 0)),
            pl.BlockSpec((H, HQ * D), lambda i: (0, 0)),
            pl.BlockSpec((H, HKV * D), lambda i: (0, 0)),
            pl.BlockSpec((H, HKV * D), lambda i: (0, 0)),
            pl.BlockSpec((1, D), lambda i: (0, 0)),
            pl.BlockSpec((1, D), lambda i: (0, 0)),
            pl.BlockSpec((BT_QKV, D), lambda i: (i, 0)),
            pl.BlockSpec((BT_QKV, D), lambda i: (i, 0)),
        ],
        out_specs=[
            pl.BlockSpec((BT_QKV, HQ * D), lambda i: (i, 0)),
            pl.BlockSpec((BT_QKV, HKV * D), lambda i: (i, 0)),
            pl.BlockSpec((BT_QKV, HKV * D), lambda i: (i, 0)),
        ],
        out_shape=[
            jax.ShapeDtypeStruct((S, HQ * D), jnp.float32),
            jax.ShapeDtypeStruct((S, HKV * D), jnp.float32),
            jax.ShapeDtypeStruct((S, HKV * D), jnp.float32),
        ],
        compiler_params=pltpu.CompilerParams(
            dimension_semantics=("parallel",)),
        name="qkv_rope",
    )(x, wq, wk, wv, q_scale, k_scale, cos, sin)


BQ = 512
BKV = 512


def _attn_kernel(q_ref, k_ref, v_ref, o_ref, s_scr, m_scr, l_scr, acc_scr):
    i = pl.program_id(1)
    q = q_ref[...]
    nkv = S // BKV
    inv_sqrt_d = 1.0 / np.sqrt(D)

    m_scr[...] = jnp.full_like(m_scr, NEG)
    for j in range(nkv):
        sl = slice(j * BKV, (j + 1) * BKV)

        @pl.when(j <= i)
        def _(j=j, sl=sl):
            kj = k_ref[sl, :]
            s = jax.lax.dot_general(q, kj, (((1,), (1,)), ((), ())),
                                    preferred_element_type=jnp.float32,
                                    precision=HP) * inv_sqrt_d
            qpos = i * BQ + jax.lax.broadcasted_iota(jnp.int32, (BQ, BKV), 0)
            kpos = j * BKV + jax.lax.broadcasted_iota(jnp.int32, (BQ, BKV), 1)
            s = jnp.where(qpos >= kpos, s, NEG)
            s_scr[:, sl] = s
            m_scr[...] = jnp.maximum(m_scr[...],
                                     jnp.max(s, axis=-1, keepdims=True))

    l_scr[...] = jnp.zeros_like(l_scr)
    for j in range(nkv):
        sl = slice(j * BKV, (j + 1) * BKV)

        @pl.when(j <= i)
        def _(j=j, sl=sl):
            p = jnp.exp(s_scr[:, sl] - m_scr[...][:, :1])
            s_scr[:, sl] = p
            l_scr[...] = l_scr[...] + jnp.sum(p, axis=-1, keepdims=True)

    acc_scr[...] = jnp.zeros_like(acc_scr)
    for j in range(nkv):
        sl = slice(j * BKV, (j + 1) * BKV)

        @pl.when(j <= i)
        def _(j=j, sl=sl):
            probs = s_scr[:, sl] / l_scr[...][:, :1]
            acc_scr[...] = acc_scr[...] + jax.lax.dot_general(
                probs, v_ref[sl, :], (((1,), (0,)), ((), ())),
                preferred_element_type=jnp.float32, precision=HP)
    o_ref[...] = acc_scr[...]


def _attention(q, k, v):
    nq = S // BQ
    return pl.pallas_call(
        _attn_kernel,
        grid=(HQ, nq),
        in_specs=[
            pl.BlockSpec((BQ, D), lambda h, i: (i, h)),
            pl.BlockSpec((S, D), lambda h, i: (0, h // (HQ // HKV))),
            pl.BlockSpec((S, D), lambda h, i: (0, h // (HQ // HKV))),
        ],
        out_specs=pl.BlockSpec((BQ, D), lambda h, i: (i, h)),
        out_shape=jax.ShapeDtypeStruct((S, HQ * D), jnp.float32),
        scratch_shapes=[
            pltpu.VMEM((BQ, S), jnp.float32),
            pltpu.VMEM((BQ, 128), jnp.float32),
            pltpu.VMEM((BQ, 128), jnp.float32),
            pltpu.VMEM((BQ, D), jnp.float32),
        ],
        compiler_params=pltpu.CompilerParams(
            dimension_semantics=("parallel", "parallel")),
        name="attn2pass",
    )(q, k, v)


BT_PR = 256


def _proj_router_kernel(a_ref, x_ref, wo_ref, rw_ref, h_out, lg_out,
                        mask_out, sel_out):
    a = a_ref[...]
    hb = jnp.dot(a, wo_ref[...], preferred_element_type=jnp.float32,
                 precision=HP) + x_ref[...]
    h_out[...] = hb
    lg = jnp.dot(hb, rw_ref[...], preferred_element_type=jnp.float32,
                 precision=HP)
    lg_out[...] = lg
    idx = jax.lax.broadcasted_iota(jnp.int32, (BT_PR, E), 1)
    m1 = jnp.max(lg, axis=-1, keepdims=True)
    i1 = jnp.min(jnp.where(lg == m1, idx, E), axis=-1, keepdims=True)
    lg2 = jnp.where(idx == i1, NEG, lg)
    m2 = jnp.max(lg2, axis=-1, keepdims=True)
    i2 = jnp.min(jnp.where(lg2 == m2, idx, E), axis=-1, keepdims=True)
    mask_out[...] = ((idx == i1) | (idx == i2)).astype(jnp.float32)
    sel_out[...] = jnp.concatenate([i1, i2], axis=-1)


def _proj_router(attn, x, wo, router_w):
    nb = S // BT_PR
    return pl.pallas_call(
        _proj_router_kernel,
        grid=(nb,),
        in_specs=[
            pl.BlockSpec((BT_PR, HQ * D), lambda i: (i, 0)),
            pl.BlockSpec((BT_PR, H), lambda i: (i, 0)),
            pl.BlockSpec((HQ * D, H), lambda i: (0, 0)),
            pl.BlockSpec((H, E), lambda i: (0, 0)),
        ],
        out_specs=[
            pl.BlockSpec((BT_PR, H), lambda i: (i, 0)),
            pl.BlockSpec((BT_PR, E), lambda i: (i, 0)),
            pl.BlockSpec((BT_PR, E), lambda i: (i, 0)),
            pl.BlockSpec((BT_PR, 2), lambda i: (i, 0)),
        ],
        out_shape=[
            jax.ShapeDtypeStruct((S, H), jnp.float32),
            jax.ShapeDtypeStruct((S, E), jnp.float32),
            jax.ShapeDtypeStruct((S, E), jnp.float32),
            jax.ShapeDtypeStruct((S, 2), jnp.int32),
        ],
        compiler_params=pltpu.CompilerParams(
            dimension_semantics=("parallel",)),
        name="proj_router",
    )(attn, x, wo, router_w)


BR = 128
NBLK_MAX = 48
NRMAX = NBLK_MAX * BR


def _moe_ffn_kernel(be_ref, valid_ref, tok_ref, h_ref, wu_ref, bu_ref,
                    wd_ref, out_ref, x_scr):
    b = pl.program_id(0)

    @pl.when(valid_ref[b] > 0)
    def _():
        base = b * BR
        for r in range(BR):
            t = tok_ref[base + r]
            x_scr[r] = h_ref[t]
        up = jnp.dot(x_scr[...].reshape(BR, H), wu_ref[0],
                     preferred_element_type=jnp.float32) + bu_ref[0]
        up = up * jax.nn.sigmoid(up)
        out_ref[...] = jnp.dot(up.astype(jnp.bfloat16), wd_ref[0],
                               preferred_element_type=jnp.float32
                               ).astype(jnp.bfloat16)


def _moe_ffn(block_expert, block_valid, token_idx, h_bf, w_up, b_up,
             w_down):
    return pl.pallas_call(
        _moe_ffn_kernel,
        grid_spec=pltpu.PrefetchScalarGridSpec(
            num_scalar_prefetch=3,
            grid=(NBLK_MAX,),
            in_specs=[
                pl.BlockSpec((S, 1, H), lambda b, be, va, tk: (0, 0, 0)),
                pl.BlockSpec((1, H, I), lambda b, be, va, tk: (be[b], 0, 0)),
                pl.BlockSpec((1, 1, I), lambda b, be, va, tk: (be[b], 0, 0)),
                pl.BlockSpec((1, I, H), lambda b, be, va, tk: (be[b], 0, 0)),
            ],
            out_specs=pl.BlockSpec((BR, H), lambda b, be, va, tk: (b, 0)),
            scratch_shapes=[pltpu.VMEM((BR, 1, H), jnp.bfloat16)],
        ),
        out_shape=jax.ShapeDtypeStruct((NRMAX, H), jnp.bfloat16),
        compiler_params=pltpu.CompilerParams(
            dimension_semantics=("arbitrary",)),
        name="moe_grouped_ffn",
    )(block_expert, block_valid, token_idx, h_bf, w_up, b_up, w_down)


BT_CB = 128


def _combine_kernel(inv0_ref, inv1_ref, h_ref, dn_ref, mask_ref, bd_ref,
                    o_ref, g0_scr, g1_scr):
    i = pl.program_id(0)
    base = i * BT_CB
    for r in range(BT_CB):
        g0_scr[r] = dn_ref[inv0_ref[base + r]]
        g1_scr[r] = dn_ref[inv1_ref[base + r]]
    o_ref[...] = (h_ref[...]
                  + g0_scr[...].reshape(BT_CB, H).astype(jnp.float32)
                  + g1_scr[...].reshape(BT_CB, H).astype(jnp.float32)
                  + jnp.dot(mask_ref[...], bd_ref[...],
                            preferred_element_type=jnp.float32))


def _combine(inv0, inv1, h, down, mask, b_down):
    nb = S // BT_CB
    return pl.pallas_call(
        _combine_kernel,
        grid_spec=pltpu.PrefetchScalarGridSpec(
            num_scalar_prefetch=2,
            grid=(nb,),
            in_specs=[
                pl.BlockSpec((BT_CB, H), lambda i, v0, v1: (i, 0)),
                pl.BlockSpec((NRMAX, 1, H), lambda i, v0, v1: (0, 0, 0)),
                pl.BlockSpec((BT_CB, E), lambda i, v0, v1: (i, 0)),
                pl.BlockSpec((E, H), lambda i, v0, v1: (0, 0)),
            ],
            out_specs=pl.BlockSpec((BT_CB, H), lambda i, v0, v1: (i, 0)),
            scratch_shapes=[pltpu.VMEM((BT_CB, 1, H), jnp.bfloat16),
                            pltpu.VMEM((BT_CB, 1, H), jnp.bfloat16)],
        ),
        out_shape=jax.ShapeDtypeStruct((S, H), jnp.float32),
        compiler_params=pltpu.CompilerParams(
            dimension_semantics=("arbitrary",)),
        name="moe_combine",
    )(inv0, inv1, h, down, mask, b_down)


def _routing_metadata(sel, mask):
    pair_expert = sel.reshape(2 * S)
    order = jnp.argsort(pair_expert, stable=True)
    inv_order = jnp.zeros((2 * S,), jnp.int32).at[order].set(
        jnp.arange(2 * S, dtype=jnp.int32))
    counts = jnp.sum(mask, axis=0).astype(jnp.int32)
    nb_e = (counts + BR - 1) // BR
    nb_cum = jnp.cumsum(nb_e)
    nblk = nb_cum[-1]
    group_start = jnp.cumsum(counts) - counts
    padded_start = (nb_cum - nb_e) * BR

    bidx = jnp.arange(NBLK_MAX, dtype=jnp.int32)
    bclamp = jnp.minimum(bidx, nblk - 1)
    e_of_b = jnp.searchsorted(nb_cum, bclamp, side="right").astype(jnp.int32)
    block_valid = (bidx < nblk).astype(jnp.int32)
    i_in_e = bclamp - (nb_cum[e_of_b] - nb_e[e_of_b])

    rowg = (group_start[e_of_b][:, None] + i_in_e[:, None] * BR
            + jnp.arange(BR, dtype=jnp.int32)[None, :])
    group_end = group_start + counts
    rvalid = rowg < group_end[e_of_b][:, None]
    rowg = jnp.where(rvalid, rowg, 0)
    token_idx = jnp.where(
        rvalid.reshape(-1), order[rowg.reshape(-1)].astype(jnp.int32) // 2, 0
    ).astype(jnp.int32)

    pe = pair_expert
    pos_padded = (padded_start[pe] + inv_order - group_start[pe]
                  ).astype(jnp.int32)
    pos2 = pos_padded.reshape(S, 2)
    return e_of_b, block_valid, token_idx, pos2[:, 0], pos2[:, 1]


def kernel(hidden_states, wq, wk, wv, wo, q_scale, k_scale, router_w, w_up,
           b_up, w_down, b_down):
    x = hidden_states.reshape(S, H)
    pos = jnp.arange(S)
    inv_freq = 1.0 / (ROPE_THETA ** (jnp.arange(0, D, 2, dtype=jnp.float32)
                                     / D))
    ang = pos[:, None].astype(jnp.float32) * inv_freq[None, :]
    cos = jnp.concatenate([jnp.cos(ang), jnp.cos(ang)], -1)
    sin = jnp.concatenate([jnp.sin(ang), jnp.sin(ang)], -1)

    q, k, v = _qkv(x, wq, wk, wv, q_scale.reshape(1, D),
                   k_scale.reshape(1, D), cos, sin)
    attn = _attention(q, k, v)
    h, router_logits, mask, sel = _proj_router(attn, x, wo, router_w)
    be, bvalid, token_idx, inv0, inv1 = _routing_metadata(sel, mask)
    down = _moe_ffn(be, bvalid, token_idx,
                    h.astype(jnp.bfloat16).reshape(S, 1, H),
                    w_up.astype(jnp.bfloat16), b_up.reshape(E, 1, I),
                    w_down.astype(jnp.bfloat16))
    out = _combine(inv0, inv1, h, down.reshape(NRMAX, 1, H), mask, b_down)
    return out.reshape(B, S, H), router_logits


# speedup vs baseline: 1.0863x; 1.0863x over previous
import functools

import jax
import jax.numpy as jnp
import numpy as np
from jax.experimental import pallas as pl
from jax.experimental.pallas import tpu as pltpu

B, S, H = 1, 2048, 2048
HQ, HKV, D = 16, 4, 128
E, I = 16, 768
EPS = 1e-6
ROPE_THETA = 10000.0
NEG = float(np.finfo(np.float32).min)

HP = None

BT_QKV = 256


def _qkv_kernel(x_ref, wq_ref, wk_ref, wv_ref, qs_ref, ks_ref, cos_ref,
                sin_ref, q_out, k_out, v_out):
    xb = x_ref[...]
    cos = cos_ref[...][:, None, :]
    sin = sin_ref[...][:, None, :]

    def norm_rope(y, nh, scale):
        y = y.reshape(BT_QKV, nh, D)
        var = jnp.mean(y * y, axis=-1, keepdims=True)
        y = y * jax.lax.rsqrt(var + EPS) * scale.reshape(1, 1, D)
        half = D // 2
        rot = jnp.concatenate([-y[..., half:], y[..., :half]], axis=-1)
        y = y * cos + rot * sin
        return y.reshape(BT_QKV, nh * D)

    q = jnp.dot(xb, wq_ref[...], preferred_element_type=jnp.float32,
                precision=HP)
    q_out[...] = norm_rope(q, HQ, qs_ref[...])
    k = jnp.dot(xb, wk_ref[...], preferred_element_type=jnp.float32,
                precision=HP)
    k_out[...] = norm_rope(k, HKV, ks_ref[...])
    v_out[...] = jnp.dot(xb, wv_ref[...], preferred_element_type=jnp.float32,
                         precision=HP)


def _qkv(x, wq, wk, wv, q_scale, k_scale, cos, sin):
    nb = S // BT_QKV
    return pl.pallas_call(
        _qkv_kernel,
        grid=(nb,),
        in_specs=[
            pl.BlockSpec((BT_QKV, H), lambda i: (i, 0)),
            pl.BlockSpec((H, HQ * D), lambda i: (0, 0)),
            pl.BlockSpec((H, HKV * D), lambda i: (0, 0)),
            pl.BlockSpec((H, HKV * D), lambda i: (0, 0)),
            pl.BlockSpec((1, D), lambda i: (0, 0)),
            pl.BlockSpec((1, D), lambda i: (0, 0)),
            pl.BlockSpec((BT_QKV, D), lambda i: (i, 0)),
            pl.BlockSpec((BT_QKV, D), lambda i: (i, 0)),
        ],
        out_specs=[
            pl.BlockSpec((BT_QKV, HQ * D), lambda i: (i, 0)),
            pl.BlockSpec((BT_QKV, HKV * D), lambda i: (i, 0)),
            pl.BlockSpec((BT_QKV, HKV * D), lambda i: (i, 0)),
        ],
        out_shape=[
            jax.ShapeDtypeStruct((S, HQ * D), jnp.float32),
            jax.ShapeDtypeStruct((S, HKV * D), jnp.float32),
            jax.ShapeDtypeStruct((S, HKV * D), jnp.float32),
        ],
        compiler_params=pltpu.CompilerParams(
            dimension_semantics=("parallel",)),
        name="qkv_rope",
    )(x, wq, wk, wv, q_scale, k_scale, cos, sin)


BQ = 512
BKV = 512


def _attn_kernel(q_ref, k_ref, v_ref, o_ref, s_scr, m_scr, l_scr, acc_scr):
    i = pl.program_id(1)
    q = q_ref[...]
    nkv = S // BKV
    inv_sqrt_d = 1.0 / np.sqrt(D)

    m_scr[...] = jnp.full_like(m_scr, NEG)
    for j in range(nkv):
        sl = slice(j * BKV, (j + 1) * BKV)

        @pl.when(j <= i)
        def _(j=j, sl=sl):
            kj = k_ref[sl, :]
            s = jax.lax.dot_general(q, kj, (((1,), (1,)), ((), ())),
                                    preferred_element_type=jnp.float32,
                                    precision=HP) * inv_sqrt_d
            qpos = i * BQ + jax.lax.broadcasted_iota(jnp.int32, (BQ, BKV), 0)
            kpos = j * BKV + jax.lax.broadcasted_iota(jnp.int32, (BQ, BKV), 1)
            s = jnp.where(qpos >= kpos, s, NEG)
            s_scr[:, sl] = s
            m_scr[...] = jnp.maximum(m_scr[...],
                                     jnp.max(s, axis=-1, keepdims=True))

    l_scr[...] = jnp.zeros_like(l_scr)
    for j in range(nkv):
        sl = slice(j * BKV, (j + 1) * BKV)

        @pl.when(j <= i)
        def _(j=j, sl=sl):
            p = jnp.exp(s_scr[:, sl] - m_scr[...][:, :1])
            s_scr[:, sl] = p
            l_scr[...] = l_scr[...] + jnp.sum(p, axis=-1, keepdims=True)

    acc_scr[...] = jnp.zeros_like(acc_scr)
    for j in range(nkv):
        sl = slice(j * BKV, (j + 1) * BKV)

        @pl.when(j <= i)
        def _(j=j, sl=sl):
            probs = s_scr[:, sl] / l_scr[...][:, :1]
            acc_scr[...] = acc_scr[...] + jax.lax.dot_general(
                probs, v_ref[sl, :], (((1,), (0,)), ((), ())),
                preferred_element_type=jnp.float32, precision=HP)
    o_ref[...] = acc_scr[...]


def _attention(q, k, v):
    nq = S // BQ
    return pl.pallas_call(
        _attn_kernel,
        grid=(HQ, nq),
        in_specs=[
            pl.BlockSpec((BQ, D), lambda h, i: (i, h)),
            pl.BlockSpec((S, D), lambda h, i: (0, h // (HQ // HKV))),
            pl.BlockSpec((S, D), lambda h, i: (0, h // (HQ // HKV))),
        ],
        out_specs=pl.BlockSpec((BQ, D), lambda h, i: (i, h)),
        out_shape=jax.ShapeDtypeStruct((S, HQ * D), jnp.float32),
        scratch_shapes=[
            pltpu.VMEM((BQ, S), jnp.float32),
            pltpu.VMEM((BQ, 128), jnp.float32),
            pltpu.VMEM((BQ, 128), jnp.float32),
            pltpu.VMEM((BQ, D), jnp.float32),
        ],
        compiler_params=pltpu.CompilerParams(
            dimension_semantics=("parallel", "parallel")),
        name="attn2pass",
    )(q, k, v)


BT_PR = 256


def _proj_router_kernel(a_ref, x_ref, wo_ref, rw_ref, tri_ref, h_out,
                        lg_out, mask_out, sel_out, rank_out, cnt_out):
    i = pl.program_id(0)
    a = a_ref[...]
    hb = jnp.dot(a, wo_ref[...], preferred_element_type=jnp.float32,
                 precision=HP) + x_ref[...]
    h_out[...] = hb
    lg = jnp.dot(hb, rw_ref[...], preferred_element_type=jnp.float32,
                 precision=HP)
    lg_out[...] = lg
    idx = jax.lax.broadcasted_iota(jnp.int32, (BT_PR, E), 1)
    m1 = jnp.max(lg, axis=-1, keepdims=True)
    i1 = jnp.min(jnp.where(lg == m1, idx, E), axis=-1, keepdims=True)
    lg2 = jnp.where(idx == i1, NEG, lg)
    m2 = jnp.max(lg2, axis=-1, keepdims=True)
    i2 = jnp.min(jnp.where(lg2 == m2, idx, E), axis=-1, keepdims=True)
    is1 = idx == i1
    is2 = idx == i2
    mask = (is1 | is2).astype(jnp.float32)
    mask_out[...] = mask
    sel_out[...] = jnp.concatenate([i1, i2], axis=-1)

    @pl.when(i == 0)
    def _():
        cnt_out[...] = jnp.zeros_like(cnt_out)

    prev = cnt_out[...]
    rank = jnp.dot(tri_ref[...], mask,
                   preferred_element_type=jnp.float32) + prev
    r0 = jnp.sum(jnp.where(is1, rank, 0.0), axis=-1, keepdims=True)
    r1 = jnp.sum(jnp.where(is2, rank, 0.0), axis=-1, keepdims=True)
    rank_out[...] = jnp.concatenate([r0, r1], axis=-1)
    cnt_out[...] = prev + jnp.sum(mask, axis=0, keepdims=True)


def _proj_router(attn, x, wo, router_w, tri):
    nb = S // BT_PR
    return pl.pallas_call(
        _proj_router_kernel,
        grid=(nb,),
        in_specs=[
            pl.BlockSpec((BT_PR, HQ * D), lambda i: (i, 0)),
            pl.BlockSpec((BT_PR, H), lambda i: (i, 0)),
            pl.BlockSpec((HQ * D, H), lambda i: (0, 0)),
            pl.BlockSpec((H, E), lambda i: (0, 0)),
            pl.BlockSpec((BT_PR, BT_PR), lambda i: (0, 0)),
        ],
        out_specs=[
            pl.BlockSpec((BT_PR, H), lambda i: (i, 0)),
            pl.BlockSpec((BT_PR, E), lambda i: (i, 0)),
            pl.BlockSpec((BT_PR, E), lambda i: (i, 0)),
            pl.BlockSpec((BT_PR, 2), lambda i: (i, 0)),
            pl.BlockSpec((BT_PR, 2), lambda i: (i, 0)),
            pl.BlockSpec((1, E), lambda i: (0, 0)),
        ],
        out_shape=[
            jax.ShapeDtypeStruct((S, H), jnp.float32),
            jax.ShapeDtypeStruct((S, E), jnp.float32),
            jax.ShapeDtypeStruct((S, E), jnp.float32),
            jax.ShapeDtypeStruct((S, 2), jnp.int32),
            jax.ShapeDtypeStruct((S, 2), jnp.float32),
            jax.ShapeDtypeStruct((1, E), jnp.float32),
        ],
        compiler_params=pltpu.CompilerParams(
            dimension_semantics=("arbitrary",)),
        name="proj_router",
    )(attn, x, wo, router_w, tri)


BR = 128
NBLK_MAX = 48
NRMAX = NBLK_MAX * BR


def _moe_ffn_kernel(be_ref, valid_ref, tok_ref, h_ref, wu_ref, bu_ref,
                    wd_ref, out_ref, x_scr):
    b = pl.program_id(0)

    @pl.when(valid_ref[b] > 0)
    def _():
        base = b * BR
        for r in range(BR):
            t = tok_ref[base + r]
            x_scr[r] = h_ref[t]
        up = jnp.dot(x_scr[...].reshape(BR, H), wu_ref[0],
                     preferred_element_type=jnp.float32) + bu_ref[0]
        up = up * jax.nn.sigmoid(up)
        out_ref[...] = jnp.dot(up.astype(jnp.bfloat16), wd_ref[0],
                               preferred_element_type=jnp.float32
                               ).astype(jnp.bfloat16)


def _moe_ffn(block_expert, block_valid, token_idx, h_bf, w_up, b_up,
             w_down):
    return pl.pallas_call(
        _moe_ffn_kernel,
        grid_spec=pltpu.PrefetchScalarGridSpec(
            num_scalar_prefetch=3,
            grid=(NBLK_MAX,),
            in_specs=[
                pl.BlockSpec((S, 1, H), lambda b, be, va, tk: (0, 0, 0)),
                pl.BlockSpec((1, H, I), lambda b, be, va, tk: (be[b], 0, 0)),
                pl.BlockSpec((1, 1, I), lambda b, be, va, tk: (be[b], 0, 0)),
                pl.BlockSpec((1, I, H), lambda b, be, va, tk: (be[b], 0, 0)),
            ],
            out_specs=pl.BlockSpec((BR, H), lambda b, be, va, tk: (b, 0)),
            scratch_shapes=[pltpu.VMEM((BR, 1, H), jnp.bfloat16)],
        ),
        out_shape=jax.ShapeDtypeStruct((NRMAX, H), jnp.bfloat16),
        compiler_params=pltpu.CompilerParams(
            dimension_semantics=("arbitrary",)),
        name="moe_grouped_ffn",
    )(block_expert, block_valid, token_idx, h_bf, w_up, b_up, w_down)


BT_CB = 128


def _combine_kernel(inv0_ref, inv1_ref, h_ref, dn_ref, mask_ref, bd_ref,
                    o_ref, g0_scr, g1_scr):
    i = pl.program_id(0)
    base = i * BT_CB
    for r in range(BT_CB):
        g0_scr[r] = dn_ref[inv0_ref[base + r]]
        g1_scr[r] = dn_ref[inv1_ref[base + r]]
    o_ref[...] = (h_ref[...]
                  + g0_scr[...].reshape(BT_CB, H).astype(jnp.float32)
                  + g1_scr[...].reshape(BT_CB, H).astype(jnp.float32)
                  + jnp.dot(mask_ref[...], bd_ref[...],
                            preferred_element_type=jnp.float32))


def _combine(inv0, inv1, h, down, mask, b_down):
    nb = S // BT_CB
    return pl.pallas_call(
        _combine_kernel,
        grid_spec=pltpu.PrefetchScalarGridSpec(
            num_scalar_prefetch=2,
            grid=(nb,),
            in_specs=[
                pl.BlockSpec((BT_CB, H), lambda i, v0, v1: (i, 0)),
                pl.BlockSpec((NRMAX, 1, H), lambda i, v0, v1: (0, 0, 0)),
                pl.BlockSpec((BT_CB, E), lambda i, v0, v1: (i, 0)),
                pl.BlockSpec((E, H), lambda i, v0, v1: (0, 0)),
            ],
            out_specs=pl.BlockSpec((BT_CB, H), lambda i, v0, v1: (i, 0)),
            scratch_shapes=[pltpu.VMEM((BT_CB, 1, H), jnp.bfloat16),
                            pltpu.VMEM((BT_CB, 1, H), jnp.bfloat16)],
        ),
        out_shape=jax.ShapeDtypeStruct((S, H), jnp.float32),
        compiler_params=pltpu.CompilerParams(
            dimension_semantics=("arbitrary",)),
        name="moe_combine",
    )(inv0, inv1, h, down, mask, b_down)


def _routing_metadata(sel, rank01, counts):
    counts = counts.reshape(E)
    nb_e = jnp.floor((counts + (BR - 1)) * (1.0 / BR)).astype(jnp.int32)
    nb_cum = jnp.cumsum(nb_e)
    nblk = nb_cum[-1]
    padded_start = ((nb_cum - nb_e) * BR).astype(jnp.float32)

    eid = jnp.arange(E, dtype=jnp.int32)
    ps_pe = jnp.sum(jnp.where(sel[:, :, None] == eid[None, None, :],
                              padded_start[None, None, :], 0.0), axis=-1)
    pos2 = (ps_pe + rank01).astype(jnp.int32)

    bidx = jnp.arange(NBLK_MAX, dtype=jnp.int32)
    block_valid = (bidx < nblk).astype(jnp.int32)
    bclamp = jnp.minimum(bidx, nblk - 1)
    e_of_b = jnp.sum((nb_cum[None, :] <= bclamp[:, None]).astype(jnp.int32),
                     axis=-1)

    pos_flat = pos2.reshape(1, 2 * S).astype(jnp.int32)
    tok_flat = (jnp.arange(2 * S, dtype=jnp.int32) // 2).astype(jnp.float32)
    g = jnp.arange(NRMAX, dtype=jnp.int32)[:, None]
    token_idx = jnp.sum(
        jnp.where(pos_flat == g, tok_flat[None, :], 0.0), axis=-1
    ).astype(jnp.int32)
    return e_of_b, block_valid, token_idx, pos2[:, 0], pos2[:, 1]


def kernel(hidden_states, wq, wk, wv, wo, q_scale, k_scale, router_w, w_up,
           b_up, w_down, b_down):
    x = hidden_states.reshape(S, H)
    pos = jnp.arange(S)
    inv_freq = 1.0 / (ROPE_THETA ** (jnp.arange(0, D, 2, dtype=jnp.float32)
                                     / D))
    ang = pos[:, None].astype(jnp.float32) * inv_freq[None, :]
    cos = jnp.concatenate([jnp.cos(ang), jnp.cos(ang)], -1)
    sin = jnp.concatenate([jnp.sin(ang), jnp.sin(ang)], -1)

    tri = (jnp.arange(BT_PR)[:, None] > jnp.arange(BT_PR)[None, :]
           ).astype(jnp.float32)
    q, k, v = _qkv(x, wq, wk, wv, q_scale.reshape(1, D),
                   k_scale.reshape(1, D), cos, sin)
    attn = _attention(q, k, v)
    h, router_logits, mask, sel, rank01, counts = _proj_router(
        attn, x, wo, router_w, tri)
    be, bvalid, token_idx, inv0, inv1 = _routing_metadata(sel, rank01,
                                                          counts)
    down = _moe_ffn(be, bvalid, token_idx,
                    h.astype(jnp.bfloat16).reshape(S, 1, H),
                    w_up.astype(jnp.bfloat16), b_up.reshape(E, 1, I),
                    w_down.astype(jnp.bfloat16))
    out = _combine(inv0, inv1, h, down.reshape(NRMAX, 1, H), mask, b_down)
    return out.reshape(B, S, H), router_logits


# speedup vs baseline: 1.1529x; 1.0613x over previous
import functools

import jax
import jax.numpy as jnp
import numpy as np
from jax.experimental import pallas as pl
from jax.experimental.pallas import tpu as pltpu

B, S, H = 1, 2048, 2048
HQ, HKV, D = 16, 4, 128
E, I = 16, 768
EPS = 1e-6
ROPE_THETA = 10000.0
NEG = float(np.finfo(np.float32).min)

HP = None

BT_QKV = 256


def _qkv_kernel(x_ref, wq_ref, wk_ref, wv_ref, qs_ref, ks_ref, cos_ref,
                sin_ref, q_out, k_out, v_out):
    xb = x_ref[...]
    cos = cos_ref[...][:, None, :]
    sin = sin_ref[...][:, None, :]

    def norm_rope(y, nh, scale):
        y = y.reshape(BT_QKV, nh, D)
        var = jnp.mean(y * y, axis=-1, keepdims=True)
        y = y * jax.lax.rsqrt(var + EPS) * scale.reshape(1, 1, D)
        half = D // 2
        rot = jnp.concatenate([-y[..., half:], y[..., :half]], axis=-1)
        y = y * cos + rot * sin
        return y.reshape(BT_QKV, nh * D)

    q = jnp.dot(xb, wq_ref[...], preferred_element_type=jnp.float32,
                precision=HP)
    q_out[...] = norm_rope(q, HQ, qs_ref[...])
    k = jnp.dot(xb, wk_ref[...], preferred_element_type=jnp.float32,
                precision=HP)
    k_out[...] = norm_rope(k, HKV, ks_ref[...])
    v_out[...] = jnp.dot(xb, wv_ref[...], preferred_element_type=jnp.float32,
                         precision=HP)


def _qkv(x, wq, wk, wv, q_scale, k_scale, cos, sin):
    nb = S // BT_QKV
    return pl.pallas_call(
        _qkv_kernel,
        grid=(nb,),
        in_specs=[
            pl.BlockSpec((BT_QKV, H), lambda i: (i, 0)),
            pl.BlockSpec((H, HQ * D), lambda i: (0, 0)),
            pl.BlockSpec((H, HKV * D), lambda i: (0, 0)),
            pl.BlockSpec((H, HKV * D), lambda i: (0, 0)),
            pl.BlockSpec((1, D), lambda i: (0, 0)),
            pl.BlockSpec((1, D), lambda i: (0, 0)),
            pl.BlockSpec((BT_QKV, D), lambda i: (i, 0)),
            pl.BlockSpec((BT_QKV, D), lambda i: (i, 0)),
        ],
        out_specs=[
            pl.BlockSpec((BT_QKV, HQ * D), lambda i: (i, 0)),
            pl.BlockSpec((BT_QKV, HKV * D), lambda i: (i, 0)),
            pl.BlockSpec((BT_QKV, HKV * D), lambda i: (i, 0)),
        ],
        out_shape=[
            jax.ShapeDtypeStruct((S, HQ * D), jnp.float32),
            jax.ShapeDtypeStruct((S, HKV * D), jnp.float32),
            jax.ShapeDtypeStruct((S, HKV * D), jnp.float32),
        ],
        compiler_params=pltpu.CompilerParams(
            dimension_semantics=("parallel",)),
        name="qkv_rope",
    )(x, wq, wk, wv, q_scale, k_scale, cos, sin)


BQ = 512
BKV = 512


def _attn_kernel(q_ref, k_ref, v_ref, o_ref, s_scr, m_scr, l_scr, acc_scr):
    i = pl.program_id(1)
    q = q_ref[...]
    nkv = S // BKV
    inv_sqrt_d = 1.0 / np.sqrt(D)

    m_scr[...] = jnp.full_like(m_scr, NEG)
    for j in range(nkv):
        sl = slice(j * BKV, (j + 1) * BKV)

        @pl.when(j <= i)
        def _(j=j, sl=sl):
            kj = k_ref[sl, :]
            s = jax.lax.dot_general(q, kj, (((1,), (1,)), ((), ())),
                                    preferred_element_type=jnp.float32,
                                    precision=HP) * inv_sqrt_d
            qpos = i * BQ + jax.lax.broadcasted_iota(jnp.int32, (BQ, BKV), 0)
            kpos = j * BKV + jax.lax.broadcasted_iota(jnp.int32, (BQ, BKV), 1)
            s = jnp.where(qpos >= kpos, s, NEG)
            s_scr[:, sl] = s
            m_scr[...] = jnp.maximum(m_scr[...],
                                     jnp.max(s, axis=-1, keepdims=True))

    l_scr[...] = jnp.zeros_like(l_scr)
    acc_scr[...] = jnp.zeros_like(acc_scr)
    for j in range(nkv):
        sl = slice(j * BKV, (j + 1) * BKV)

        @pl.when(j <= i)
        def _(j=j, sl=sl):
            p = jnp.exp(s_scr[:, sl] - m_scr[...][:, :1])
            l_scr[...] = l_scr[...] + jnp.sum(p, axis=-1, keepdims=True)
            acc_scr[...] = acc_scr[...] + jax.lax.dot_general(
                p, v_ref[sl, :], (((1,), (0,)), ((), ())),
                preferred_element_type=jnp.float32, precision=HP)
    o_ref[...] = acc_scr[...] / l_scr[...]


def _attention(q, k, v):
    nq = S // BQ
    return pl.pallas_call(
        _attn_kernel,
        grid=(HQ, nq),
        in_specs=[
            pl.BlockSpec((BQ, D), lambda h, i: (i, h)),
            pl.BlockSpec((S, D), lambda h, i: (0, h // (HQ // HKV))),
            pl.BlockSpec((S, D), lambda h, i: (0, h // (HQ // HKV))),
        ],
        out_specs=pl.BlockSpec((BQ, D), lambda h, i: (i, h)),
        out_shape=jax.ShapeDtypeStruct((S, HQ * D), jnp.float32),
        scratch_shapes=[
            pltpu.VMEM((BQ, S), jnp.float32),
            pltpu.VMEM((BQ, 128), jnp.float32),
            pltpu.VMEM((BQ, 128), jnp.float32),
            pltpu.VMEM((BQ, D), jnp.float32),
        ],
        compiler_params=pltpu.CompilerParams(
            dimension_semantics=("parallel", "parallel")),
        name="attn2pass",
    )(q, k, v)


BT_PR = 256


def _proj_router_kernel(a_ref, x_ref, wo_ref, rw_ref, tri_ref, h_out,
                        lg_out, mask_out, sel_out, rank_out, cnt_out):
    i = pl.program_id(0)
    a = a_ref[...]
    hb = jnp.dot(a, wo_ref[...], preferred_element_type=jnp.float32,
                 precision=HP) + x_ref[...]
    h_out[...] = hb
    lg = jnp.dot(hb, rw_ref[...], preferred_element_type=jnp.float32,
                 precision=HP)
    lg_out[...] = lg
    idx = jax.lax.broadcasted_iota(jnp.int32, (BT_PR, E), 1)
    m1 = jnp.max(lg, axis=-1, keepdims=True)
    i1 = jnp.min(jnp.where(lg == m1, idx, E), axis=-1, keepdims=True)
    lg2 = jnp.where(idx == i1, NEG, lg)
    m2 = jnp.max(lg2, axis=-1, keepdims=True)
    i2 = jnp.min(jnp.where(lg2 == m2, idx, E), axis=-1, keepdims=True)
    is1 = idx == i1
    is2 = idx == i2
    mask = (is1 | is2).astype(jnp.float32)
    mask_out[...] = mask
    sel_out[...] = jnp.concatenate([i1, i2], axis=-1)

    @pl.when(i == 0)
    def _():
        cnt_out[...] = jnp.zeros_like(cnt_out)

    prev = cnt_out[...]
    rank = jnp.dot(tri_ref[...], mask,
                   preferred_element_type=jnp.float32) + prev
    r0 = jnp.sum(jnp.where(is1, rank, 0.0), axis=-1, keepdims=True)
    r1 = jnp.sum(jnp.where(is2, rank, 0.0), axis=-1, keepdims=True)
    rank_out[...] = jnp.concatenate([r0, r1], axis=-1)
    cnt_out[...] = prev + jnp.sum(mask, axis=0, keepdims=True)


def _proj_router(attn, x, wo, router_w, tri):
    nb = S // BT_PR
    return pl.pallas_call(
        _proj_router_kernel,
        grid=(nb,),
        in_specs=[
            pl.BlockSpec((BT_PR, HQ * D), lambda i: (i, 0)),
            pl.BlockSpec((BT_PR, H), lambda i: (i, 0)),
            pl.BlockSpec((HQ * D, H), lambda i: (0, 0)),
            pl.BlockSpec((H, E), lambda i: (0, 0)),
            pl.BlockSpec((BT_PR, BT_PR), lambda i: (0, 0)),
        ],
        out_specs=[
            pl.BlockSpec((BT_PR, H), lambda i: (i, 0)),
            pl.BlockSpec((BT_PR, E), lambda i: (i, 0)),
            pl.BlockSpec((BT_PR, E), lambda i: (i, 0)),
            pl.BlockSpec((BT_PR, 2), lambda i: (i, 0)),
            pl.BlockSpec((BT_PR, 2), lambda i: (i, 0)),
            pl.BlockSpec((1, E), lambda i: (0, 0)),
        ],
        out_shape=[
            jax.ShapeDtypeStruct((S, H), jnp.float32),
            jax.ShapeDtypeStruct((S, E), jnp.float32),
            jax.ShapeDtypeStruct((S, E), jnp.float32),
            jax.ShapeDtypeStruct((S, 2), jnp.int32),
            jax.ShapeDtypeStruct((S, 2), jnp.float32),
            jax.ShapeDtypeStruct((1, E), jnp.float32),
        ],
        compiler_params=pltpu.CompilerParams(
            dimension_semantics=("arbitrary",)),
        name="proj_router",
    )(attn, x, wo, router_w, tri)


BR = 128
NBLK_MAX = 48
NRMAX = NBLK_MAX * BR


def _moe_ffn_kernel(be_ref, valid_ref, tok_ref, h_ref, wu_ref, bu_ref,
                    wd_ref, out_ref, x_scr):
    b = pl.program_id(0)

    @pl.when(valid_ref[b] > 0)
    def _():
        base = b * BR
        for r in range(BR):
            t = tok_ref[base + r]
            x_scr[r] = h_ref[t]
        up = jnp.dot(x_scr[...].reshape(BR, H), wu_ref[0],
                     preferred_element_type=jnp.float32) + bu_ref[0]
        up = up * jax.nn.sigmoid(up)
        out_ref[...] = jnp.dot(up.astype(jnp.bfloat16), wd_ref[0],
                               preferred_element_type=jnp.float32
                               ).astype(jnp.bfloat16)


def _moe_ffn(block_expert, block_valid, token_idx, h_bf, w_up, b_up,
             w_down):
    return pl.pallas_call(
        _moe_ffn_kernel,
        grid_spec=pltpu.PrefetchScalarGridSpec(
            num_scalar_prefetch=3,
            grid=(NBLK_MAX,),
            in_specs=[
                pl.BlockSpec((S, 1, H), lambda b, be, va, tk: (0, 0, 0)),
                pl.BlockSpec((1, H, I), lambda b, be, va, tk: (be[b], 0, 0)),
                pl.BlockSpec((1, 1, I), lambda b, be, va, tk: (be[b], 0, 0)),
                pl.BlockSpec((1, I, H), lambda b, be, va, tk: (be[b], 0, 0)),
            ],
            out_specs=pl.BlockSpec((BR, H), lambda b, be, va, tk: (b, 0)),
            scratch_shapes=[pltpu.VMEM((BR, 1, H), jnp.bfloat16)],
        ),
        out_shape=jax.ShapeDtypeStruct((NRMAX, H), jnp.bfloat16),
        compiler_params=pltpu.CompilerParams(
            dimension_semantics=("arbitrary",)),
        name="moe_grouped_ffn",
    )(block_expert, block_valid, token_idx, h_bf, w_up, b_up, w_down)


BT_CB = 128


def _combine_kernel(inv0_ref, inv1_ref, h_ref, dn_ref, mask_ref, bd_ref,
                    o_ref, g0_scr, g1_scr):
    i = pl.program_id(0)
    base = i * BT_CB
    for r in range(BT_CB):
        g0_scr[r] = dn_ref[inv0_ref[base + r]]
        g1_scr[r] = dn_ref[inv1_ref[base + r]]
    o_ref[...] = (h_ref[...]
                  + g0_scr[...].reshape(BT_CB, H).astype(jnp.float32)
                  + g1_scr[...].reshape(BT_CB, H).astype(jnp.float32)
                  + jnp.dot(mask_ref[...], bd_ref[...],
                            preferred_element_type=jnp.float32))


def _combine(inv0, inv1, h, down, mask, b_down):
    nb = S // BT_CB
    return pl.pallas_call(
        _combine_kernel,
        grid_spec=pltpu.PrefetchScalarGridSpec(
            num_scalar_prefetch=2,
            grid=(nb,),
            in_specs=[
                pl.BlockSpec((BT_CB, H), lambda i, v0, v1: (i, 0)),
                pl.BlockSpec((NRMAX, 1, H), lambda i, v0, v1: (0, 0, 0)),
                pl.BlockSpec((BT_CB, E), lambda i, v0, v1: (i, 0)),
                pl.BlockSpec((E, H), lambda i, v0, v1: (0, 0)),
            ],
            out_specs=pl.BlockSpec((BT_CB, H), lambda i, v0, v1: (i, 0)),
            scratch_shapes=[pltpu.VMEM((BT_CB, 1, H), jnp.bfloat16),
                            pltpu.VMEM((BT_CB, 1, H), jnp.bfloat16)],
        ),
        out_shape=jax.ShapeDtypeStruct((S, H), jnp.float32),
        compiler_params=pltpu.CompilerParams(
            dimension_semantics=("arbitrary",)),
        name="moe_combine",
    )(inv0, inv1, h, down, mask, b_down)


def _routing_metadata(sel, rank01, counts):
    counts = counts.reshape(E)
    nb_e = jnp.floor((counts + (BR - 1)) * (1.0 / BR)).astype(jnp.int32)
    nb_cum = jnp.cumsum(nb_e)
    nblk = nb_cum[-1]
    padded_start = ((nb_cum - nb_e) * BR).astype(jnp.float32)

    eid = jnp.arange(E, dtype=jnp.int32)
    ps_pe = jnp.sum(jnp.where(sel[:, :, None] == eid[None, None, :],
                              padded_start[None, None, :], 0.0), axis=-1)
    pos2 = (ps_pe + rank01).astype(jnp.int32)

    bidx = jnp.arange(NBLK_MAX, dtype=jnp.int32)
    block_valid = (bidx < nblk).astype(jnp.int32)
    bclamp = jnp.minimum(bidx, nblk - 1)
    e_of_b = jnp.sum((nb_cum[None, :] <= bclamp[:, None]).astype(jnp.int32),
                     axis=-1)

    pos_flat = pos2.reshape(1, 2 * S).astype(jnp.int32)
    tok_flat = (jnp.arange(2 * S, dtype=jnp.int32) // 2).astype(jnp.float32)
    g = jnp.arange(NRMAX, dtype=jnp.int32)[:, None]
    token_idx = jnp.sum(
        jnp.where(pos_flat == g, tok_flat[None, :], 0.0), axis=-1
    ).astype(jnp.int32)
    return e_of_b, block_valid, token_idx, pos2[:, 0], pos2[:, 1]


def kernel(hidden_states, wq, wk, wv, wo, q_scale, k_scale, router_w, w_up,
           b_up, w_down, b_down):
    x = hidden_states.reshape(S, H)
    pos = jnp.arange(S)
    inv_freq = 1.0 / (ROPE_THETA ** (jnp.arange(0, D, 2, dtype=jnp.float32)
                                     / D))
    ang = pos[:, None].astype(jnp.float32) * inv_freq[None, :]
    cos = jnp.concatenate([jnp.cos(ang), jnp.cos(ang)], -1)
    sin = jnp.concatenate([jnp.sin(ang), jnp.sin(ang)], -1)

    tri = (jnp.arange(BT_PR)[:, None] > jnp.arange(BT_PR)[None, :]
           ).astype(jnp.float32)
    q, k, v = _qkv(x, wq, wk, wv, q_scale.reshape(1, D),
                   k_scale.reshape(1, D), cos, sin)
    attn = _attention(q, k, v)
    h, router_logits, mask, sel, rank01, counts = _proj_router(
        attn, x, wo, router_w, tri)
    be, bvalid, token_idx, inv0, inv1 = _routing_metadata(sel, rank01,
                                                          counts)
    down = _moe_ffn(be, bvalid, token_idx,
                    h.astype(jnp.bfloat16).reshape(S, 1, H),
                    w_up.astype(jnp.bfloat16), b_up.reshape(E, 1, I),
                    w_down.astype(jnp.bfloat16))
    out = _combine(inv0, inv1, h, down.reshape(NRMAX, 1, H), mask, b_down)
    return out.reshape(B, S, H), router_logits
